# baseline (speedup 1.0000x reference)
import jax
import jax.numpy as jnp
from jax import lax
from jax.experimental import pallas as pl
from jax.experimental.pallas import tpu as pltpu
from jax.experimental.pallas import tpu_sc as plsc

_B = 16384
_F = 26
_V = 100000
_D = 32
_NSUB = 4
_VSUB = _V // _NSUB
_GRP = _B // (16 * 8)


def _body(x_hbm, tab_hbm, out_hbm, row_v, buf_v, rsem):
    d = lax.axis_index("s") * 2 + lax.axis_index("c")

    def per_field(f, carry):
        row_cp = pltpu.async_copy(tab_hbm.at[f, d], row_v, rsem)
        pltpu.sync_copy(x_hbm.at[f], buf_v)
        row_cp.wait()

        def gather8(i, carry2):
            base = i * 128
            for u in range(8):
                sl = pl.ds(base + u * 16, 16)
                iv = plsc.bitcast(buf_v[sl], jnp.int32)
                buf_v[sl] = plsc.load_gather(row_v, [iv])
            return carry2

        lax.fori_loop(0, _GRP, gather8, 0)
        pltpu.sync_copy(buf_v, out_hbm.at[f, d])
        return carry

    lax.fori_loop(0, _F, per_field, 0)


@jax.jit
def kernel(x_cat, tables):
    x_t = jax.lax.bitcast_convert_type(x_cat.T, jnp.float32)
    tab_t = jnp.transpose(tables, (0, 2, 1))
    mesh = plsc.VectorSubcoreMesh(core_axis_name="c", subcore_axis_name="s")
    out = pl.kernel(
        _body,
        mesh=mesh,
        out_type=jax.ShapeDtypeStruct((_F, _D, _B), jnp.float32),
        scratch_types=[
            pltpu.VMEM((_V,), jnp.float32),
            pltpu.VMEM((_B,), jnp.float32),
            pltpu.SemaphoreType.DMA,
        ],
        compiler_params=pltpu.CompilerParams(
            use_tc_tiling_on_sc=True, needs_layout_passes=False
        ),
    )(x_t, tab_t)
    return jnp.transpose(out, (2, 0, 1))


# speedup vs baseline: 1.0484x; 1.0484x over previous
import jax
import jax.numpy as jnp
from jax import lax
from jax.experimental import pallas as pl
from jax.experimental.pallas import tpu as pltpu
from jax.experimental.pallas import tpu_sc as plsc

_B = 16384
_F = 26
_V = 100000
_D = 32
_NSUB = 4
_VSUB = _V // _NSUB
_GRP = _B // (16 * 8)


def _body(x_hbm, tab_hbm, out_hbm, row_v, buf_v, rsem):
    d = lax.axis_index("s") * 2 + lax.axis_index("c")

    pltpu.async_copy(tab_hbm.at[0, d], row_v, rsem)

    def per_field(f, carry):
        pltpu.sync_copy(x_hbm.at[f], buf_v)
        pltpu.make_async_copy(tab_hbm.at[0, d], row_v, rsem).wait()

        def gather8(i, carry2):
            base = i * 128
            for u in range(8):
                sl = pl.ds(base + u * 16, 16)
                iv = plsc.bitcast(buf_v[sl], jnp.int32)
                buf_v[sl] = plsc.load_gather(row_v, [iv])
            return carry2

        lax.fori_loop(0, _GRP, gather8, 0)

        @pl.when(f + 1 < _F)
        def _():
            pltpu.async_copy(tab_hbm.at[f + 1, d], row_v, rsem)

        pltpu.sync_copy(buf_v, out_hbm.at[f, d])
        return carry

    lax.fori_loop(0, _F, per_field, 0)


@jax.jit
def kernel(x_cat, tables):
    x_t = jax.lax.bitcast_convert_type(x_cat.T, jnp.float32)
    tab_t = jnp.transpose(tables, (0, 2, 1))
    mesh = plsc.VectorSubcoreMesh(core_axis_name="c", subcore_axis_name="s")
    out = pl.kernel(
        _body,
        mesh=mesh,
        out_type=jax.ShapeDtypeStruct((_F, _D, _B), jnp.float32),
        scratch_types=[
            pltpu.VMEM((_V,), jnp.float32),
            pltpu.VMEM((_B,), jnp.float32),
            pltpu.SemaphoreType.DMA,
        ],
        compiler_params=pltpu.CompilerParams(
            use_tc_tiling_on_sc=True, needs_layout_passes=False
        ),
    )(x_t, tab_t)
    return jnp.transpose(out, (2, 0, 1))
